# baseline (speedup 1.0000x reference)
import functools

import jax
import jax.numpy as jnp
from jax import lax
from jax.experimental import pallas as pl
from jax.experimental.pallas import tpu as pltpu
from jax.experimental.pallas import tpu_sc as plsc

N = 10000
E = 320000
D = 128
NT = 100
NRBF = 16

NC = 2
NS = 16
NW = NC * NS
EPT = E // NW
ROWS = 80
ACC = 237 * 128
BE = 6400
NB = E // BE
BN = 2000
NBN = N // BN

_mesh = plsc.VectorSubcoreMesh(core_axis_name="c", subcore_axis_name="s")
_sc_params = pltpu.CompilerParams(needs_layout_passes=False)


@functools.partial(
    pl.kernel,
    mesh=_mesh,
    compiler_params=_sc_params,
    out_type=[
        jax.ShapeDtypeStruct((E,), jnp.float32),
        jax.ShapeDtypeStruct((E,), jnp.float32),
        jax.ShapeDtypeStruct((E,), jnp.int32),
    ],
    scratch_types=[
        pltpu.VMEM((3 * N,), jnp.float32),
        pltpu.VMEM((N,), jnp.int32),
        pltpu.VMEM((EPT,), jnp.int32),
        pltpu.VMEM((EPT,), jnp.int32),
        pltpu.VMEM((3 * EPT,), jnp.float32),
        pltpu.VMEM((EPT,), jnp.float32),
        pltpu.VMEM((EPT,), jnp.float32),
        pltpu.VMEM((EPT,), jnp.int32),
        pltpu.VMEM((16,), jnp.float32),
        pltpu.SemaphoreType.DMA,
    ],
)
def _geom(posf_h, z_hbm, eif, shf_h, w16,
          s_out, d_out, zs_out,
          pf, zn, sv, dv, shf, sb, db, zb, wv, sem):
    wid = lax.axis_index("s") * NC + lax.axis_index("c")
    base = wid * EPT
    descs = [
        pltpu.async_copy(posf_h, pf, sem),
        pltpu.async_copy(z_hbm, zn, sem),
        pltpu.async_copy(eif.at[pl.ds(base, EPT)], sv, sem),
        pltpu.async_copy(eif.at[pl.ds(E + base, EPT)], dv, sem),
        pltpu.async_copy(shf_h.at[pl.ds(3 * base, 3 * EPT)], shf, sem),
        pltpu.async_copy(w16, wv, sem),
    ]
    for de in descs:
        de.wait()
    one = jnp.full((16,), 1, jnp.int32)
    w1 = plsc.load_gather(wv, [one])
    w2 = plsc.load_gather(wv, [one + 1])
    w3 = plsc.load_gather(wv, [one + 2])
    iota3 = 3 * jnp.arange(16, dtype=jnp.int32)

    @pl.loop(0, EPT // 16, unroll=4)
    def body(i):
        off = i * 16
        s16 = sv[pl.ds(off, 16)] * 3
        d16 = dv[pl.ds(off, 16)] * 3
        shi = off * 3 + iota3
        ax = (plsc.load_gather(pf, [d16]) - plsc.load_gather(pf, [s16])
              + plsc.load_gather(shf, [shi]))
        ay = (plsc.load_gather(pf, [d16 + 1]) - plsc.load_gather(pf, [s16 + 1])
              + plsc.load_gather(shf, [shi + 1]))
        az = (plsc.load_gather(pf, [d16 + 2]) - plsc.load_gather(pf, [s16 + 2])
              + plsc.load_gather(shf, [shi + 2]))
        zg = plsc.load_gather(zn, [sv[pl.ds(off, 16)]])
        sb[pl.ds(off, 16)] = ax * ax + ay * ay + az * az
        db[pl.ds(off, 16)] = ax * w1 + ay * w2 + az * w3
        zb[pl.ds(off, 16)] = zg

    outs = [
        pltpu.async_copy(sb, s_out.at[pl.ds(base, EPT)], sem),
        pltpu.async_copy(db, d_out.at[pl.ds(base, EPT)], sem),
        pltpu.async_copy(zb, zs_out.at[pl.ds(base, EPT)], sem),
    ]
    for de in outs:
        de.wait()


def _silu(x):
    return 0.5 * x + (0.5 * x) * jnp.tanh(0.5 * x)


def _edge_body(s_ref, d_ref, zs_ref, W3T_ref, embT_ref, b3c_ref, W1cat_ref,
               b1_ref, cen_ref, w0_ref, W5T_ref, ox_ref, oy_ref, oz_ref,
               nfs_ref):
    i = pl.program_id(0)

    @pl.when(i == 0)
    def _():
        x = jnp.dot(W3T_ref[...], embT_ref[...],
                    preferred_element_type=jnp.float32) + b3c_ref[...]
        nfT = _silu(x)
        hi = nfT.astype(jnp.bfloat16)
        lo = (nfT - hi.astype(jnp.float32)).astype(jnp.bfloat16)
        nfs_ref[0:D, :] = hi
        nfs_ref[D:2 * D, :] = lo

    s = s_ref[0, :]
    dd = d_ref[0, :]
    zs = zs_ref[0, :]
    se = s + 1e-12
    inv = jax.lax.rsqrt(se)
    ln = se * inv
    shw = w0_ref[0, 0] + dd * inv
    diff = cen_ref[...] - ln[None, :]
    rbf = jnp.exp(-2.0 * diff * diff)
    rh = rbf.astype(jnp.bfloat16)
    rl = (rbf - rh.astype(jnp.float32)).astype(jnp.bfloat16)
    rcat = jnp.concatenate([rh, rl], axis=0)
    pre = jnp.dot(W1cat_ref[...], rcat, preferred_element_type=jnp.float32)
    pre = pre + b1_ref[...]
    ef = _silu(pre)
    ohT = (jax.lax.broadcasted_iota(jnp.int32, (D, BE), 0) == zs[None, :])
    ohb = ohT.astype(jnp.bfloat16)
    sel = jnp.dot(nfs_ref[...], ohb, preferred_element_type=jnp.float32)
    nfT = sel[0:D, :] + sel[D:2 * D, :]
    t = nfT * ef * shw[None, :]
    out = jnp.dot(W5T_ref[...], t, preferred_element_type=jnp.float32)
    ox_ref[...] = out[0:1, :]
    oy_ref[...] = out[1:2, :]
    oz_ref[...] = out[2:3, :]


_edge = pl.pallas_call(
    _edge_body,
    grid=(NB,),
    in_specs=[
        pl.BlockSpec((1, BE), lambda i: (0, i)),
        pl.BlockSpec((1, BE), lambda i: (0, i)),
        pl.BlockSpec((1, BE), lambda i: (0, i)),
        pl.BlockSpec((D, D), lambda i: (0, 0)),
        pl.BlockSpec((D, D), lambda i: (0, 0)),
        pl.BlockSpec((D, 1), lambda i: (0, 0)),
        pl.BlockSpec((D, 2 * NRBF), lambda i: (0, 0)),
        pl.BlockSpec((D, 1), lambda i: (0, 0)),
        pl.BlockSpec((NRBF, 1), lambda i: (0, 0)),
        pl.BlockSpec((1, 1), lambda i: (0, 0)),
        pl.BlockSpec((3, D), lambda i: (0, 0)),
    ],
    out_specs=[
        pl.BlockSpec((1, BE), lambda i: (0, i)),
        pl.BlockSpec((1, BE), lambda i: (0, i)),
        pl.BlockSpec((1, BE), lambda i: (0, i)),
    ],
    out_shape=[
        jax.ShapeDtypeStruct((1, E), jnp.float32),
        jax.ShapeDtypeStruct((1, E), jnp.float32),
        jax.ShapeDtypeStruct((1, E), jnp.float32),
    ],
    scratch_shapes=[pltpu.VMEM((2 * D, D), jnp.bfloat16)],
)


@functools.partial(
    pl.kernel,
    mesh=_mesh,
    compiler_params=_sc_params,
    out_type=jax.ShapeDtypeStruct((NC * ACC,), jnp.float32),
    scratch_types=[
        pltpu.VMEM((EPT,), jnp.int32),
        pltpu.VMEM((ROWS * 128,), jnp.float32),
        pltpu.VMEM((ROWS * 128,), jnp.float32),
        pltpu.VMEM((ROWS * 128,), jnp.float32),
        pltpu.VMEM((ROWS, 128), jnp.int32),
        pltpu.VMEM((ROWS, 128), jnp.int32),
        pltpu.VMEM((ROWS, 128), jnp.int32),
        pltpu.VMEM((ACC,), jnp.float32),
        pltpu.VMEM_SHARED((ACC,), jnp.float32),
        pltpu.SemaphoreType.DMA,
    ],
)
def _scat(eif, oxh, oyh, ozh, zeros_hbm, out_hbm,
          dv, ux, uy, uz, ix, iy, iz, bb, acc, sem):
    cid = lax.axis_index("c")
    sid = lax.axis_index("s")
    wid = sid * NC + cid
    base = wid * EPT
    pltpu.sync_copy(eif.at[pl.ds(E + base, EPT)], dv)
    ins = [
        pltpu.async_copy(oxh.at[pl.ds(base, EPT)], ux.at[pl.ds(0, EPT)], sem),
        pltpu.async_copy(oyh.at[pl.ds(base, EPT)], uy.at[pl.ds(0, EPT)], sem),
        pltpu.async_copy(ozh.at[pl.ds(base, EPT)], uz.at[pl.ds(0, EPT)], sem),
    ]

    @pl.when(sid == 0)
    def _():
        pltpu.sync_copy(zeros_hbm, bb)
        pltpu.sync_copy(bb, acc)

    @pl.loop(0, ROWS - 2, unroll=4)
    def ibody(r):
        for k in range(8):
            d16 = dv[pl.ds(r * 128 + k * 16, 16)]
            f = d16 * 3
            ix[r, pl.ds(k * 16, 16)] = f
            iy[r, pl.ds(k * 16, 16)] = f + 1
            iz[r, pl.ds(k * 16, 16)] = f + 2

    d16 = dv[pl.ds((ROWS - 2) * 128, 16)]
    f = d16 * 3
    ix[ROWS - 2, pl.ds(0, 16)] = f
    iy[ROWS - 2, pl.ds(0, 16)] = f + 1
    iz[ROWS - 2, pl.ds(0, 16)] = f + 2
    ar16 = jnp.arange(16, dtype=jnp.int32)
    for k in range(1, 8):
        pad = 3 * N + (k - 1) * 16 + ar16
        ix[ROWS - 2, pl.ds(k * 16, 16)] = pad
        iy[ROWS - 2, pl.ds(k * 16, 16)] = pad
        iz[ROWS - 2, pl.ds(k * 16, 16)] = pad
    for k in range(8):
        pad = 3 * N + 112 + k * 16 + ar16
        ix[ROWS - 1, pl.ds(k * 16, 16)] = pad
        iy[ROWS - 1, pl.ds(k * 16, 16)] = pad
        iz[ROWS - 1, pl.ds(k * 16, 16)] = pad

    for de in ins:
        de.wait()
    zero16 = jnp.zeros((16,), jnp.float32)
    for k in range((ROWS * 128 - EPT) // 16):
        off = EPT + k * 16
        ux[pl.ds(off, 16)] = zero16
        uy[pl.ds(off, 16)] = zero16
        uz[pl.ds(off, 16)] = zero16

    plsc.subcore_barrier()

    @pl.loop(0, ROWS // 8)
    def sbody(g):
        descs = []
        for rr in range(8):
            r = g * 8 + rr
            for u, ii in ((ux, ix), (uy, iy), (uz, iz)):
                descs.append(pltpu.async_copy(
                    u.at[pl.ds(r * 128, 128)], acc.at[ii.at[r]], sem,
                    add=True))
        for de in descs:
            de.wait()

    plsc.subcore_barrier()

    @pl.when(sid == 0)
    def _():
        pltpu.sync_copy(acc, bb)
        pltpu.sync_copy(bb, out_hbm.at[pl.ds(cid * ACC, ACC)])


def _fin_body(p_ref, z_ref, emb_ref, W4_ref, W5_ref, b5_ref, o_ref, w45_ref):
    i = pl.program_id(0)

    @pl.when(i == 0)
    def _():
        w45 = jnp.dot(W4_ref[...], W5_ref[...],
                      preferred_element_type=jnp.float32)
        w45_ref[...] = jnp.dot(emb_ref[...], w45,
                               preferred_element_type=jnp.float32)

    zb = z_ref[0, 0, :]
    oh = (jax.lax.broadcasted_iota(jnp.int32, (BN, D), 1) == zb[:, None])
    bse = jnp.dot(oh.astype(jnp.float32), w45_ref[...],
                  preferred_element_type=jnp.float32)
    o_ref[...] = p_ref[0] + p_ref[1] + bse + b5_ref[...][None, :]


_fin = pl.pallas_call(
    _fin_body,
    grid=(NBN,),
    in_specs=[
        pl.BlockSpec((2, BN, 3), lambda i: (0, i, 0)),
        pl.BlockSpec((1, 1, BN), lambda i: (i, 0, 0)),
        pl.BlockSpec((D, D), lambda i: (0, 0)),
        pl.BlockSpec((D, D), lambda i: (0, 0)),
        pl.BlockSpec((D, 3), lambda i: (0, 0)),
        pl.BlockSpec((3,), lambda i: (0,)),
    ],
    out_specs=pl.BlockSpec((BN, 3), lambda i: (i, 0)),
    out_shape=jax.ShapeDtypeStruct((N, 3), jnp.float32),
    scratch_shapes=[pltpu.VMEM((D, 3), jnp.float32)],
)


def kernel(z, pos, edge_index, shift_vector, emb, rbf_centers, W1, b1,
           W3, b3, w_sh, W4, W5, b5):
    f32 = jnp.float32
    bf16 = jnp.bfloat16
    w16 = jnp.pad(w_sh, (0, 12))
    emb128 = jnp.pad(emb, ((0, D - NT), (0, 0)))
    W1T = W1.T
    W1h = W1T.astype(bf16)
    W1l = (W1T - W1h.astype(f32)).astype(bf16)
    W1cat = jnp.concatenate([W1h, W1l], axis=1)
    eif = edge_index.reshape(2 * E)
    s_e, d_e, zs = _geom(pos.reshape(3 * N), z, eif,
                         shift_vector.reshape(3 * E), w16)
    o3x, o3y, o3z = _edge(s_e.reshape(1, E), d_e.reshape(1, E),
                          zs.reshape(1, E),
                          W3.T, emb128.T, b3.reshape(D, 1), W1cat,
                          b1.reshape(D, 1), rbf_centers.reshape(NRBF, 1),
                          w_sh[0].reshape(1, 1), W5.T)
    zeros = jnp.zeros((ACC,), f32)
    parts = _scat(eif, o3x.reshape(E), o3y.reshape(E), o3z.reshape(E), zeros)
    p3 = parts.reshape(NC, ACC)[:, :3 * N].reshape(NC, N, 3)
    return _fin(p3, z.reshape(NBN, 1, BN), emb128, W4, W5, b5)


# speedup vs baseline: 1.8300x; 1.8300x over previous
import functools

import jax
import jax.numpy as jnp
from jax import lax
from jax.experimental import pallas as pl
from jax.experimental.pallas import tpu as pltpu
from jax.experimental.pallas import tpu_sc as plsc

N = 10000
E = 320000
D = 128
NT = 100
NRBF = 16

NC = 2
NS = 16
NW = NC * NS
EPT = E // NW
ROWS = 80
ACC = 237 * 128
BE = 6400
NB = E // BE
BN = 2000
NBN = N // BN

_mesh = plsc.VectorSubcoreMesh(core_axis_name="c", subcore_axis_name="s")
_sc_params = pltpu.CompilerParams(needs_layout_passes=False)


@functools.partial(
    pl.kernel,
    mesh=_mesh,
    compiler_params=_sc_params,
    out_type=[
        jax.ShapeDtypeStruct((E,), jnp.float32),
        jax.ShapeDtypeStruct((E,), jnp.float32),
        jax.ShapeDtypeStruct((E,), jnp.int32),
    ],
    scratch_types=[
        pltpu.VMEM((N,), jnp.float32),
        pltpu.VMEM((N,), jnp.float32),
        pltpu.VMEM((N,), jnp.float32),
        pltpu.VMEM((N,), jnp.int32),
        pltpu.VMEM((EPT,), jnp.int32),
        pltpu.VMEM((EPT,), jnp.int32),
        pltpu.VMEM((EPT,), jnp.float32),
        pltpu.VMEM((EPT,), jnp.float32),
        pltpu.VMEM((EPT,), jnp.float32),
        pltpu.VMEM((EPT,), jnp.float32),
        pltpu.VMEM((EPT,), jnp.float32),
        pltpu.VMEM((EPT,), jnp.int32),
        pltpu.VMEM((16,), jnp.float32),
        pltpu.SemaphoreType.DMA,
    ],
)
def _geom(pxh, pyh, pzh, z_hbm, eif, sxh, syh, szh, w16,
          s_out, d_out, zs_out,
          px, py, pz, zn, sv, dv, sx, sy, sz, sb, db, zb, wv, sem):
    wid = lax.axis_index("s") * NC + lax.axis_index("c")
    base = wid * EPT
    descs = [
        pltpu.async_copy(pxh, px, sem),
        pltpu.async_copy(pyh, py, sem),
        pltpu.async_copy(pzh, pz, sem),
        pltpu.async_copy(z_hbm, zn, sem),
        pltpu.async_copy(eif.at[pl.ds(base, EPT)], sv, sem),
        pltpu.async_copy(eif.at[pl.ds(E + base, EPT)], dv, sem),
        pltpu.async_copy(sxh.at[pl.ds(base, EPT)], sx, sem),
        pltpu.async_copy(syh.at[pl.ds(base, EPT)], sy, sem),
        pltpu.async_copy(szh.at[pl.ds(base, EPT)], sz, sem),
        pltpu.async_copy(w16, wv, sem),
    ]
    for de in descs:
        de.wait()
    one = jnp.full((16,), 1, jnp.int32)
    w1 = plsc.load_gather(wv, [one])
    w2 = plsc.load_gather(wv, [one + 1])
    w3 = plsc.load_gather(wv, [one + 2])

    @pl.loop(0, EPT // 16, unroll=4)
    def body(i):
        off = i * 16
        s16 = sv[pl.ds(off, 16)]
        d16 = dv[pl.ds(off, 16)]
        ax = plsc.load_gather(px, [d16]) - plsc.load_gather(px, [s16]) + sx[pl.ds(off, 16)]
        ay = plsc.load_gather(py, [d16]) - plsc.load_gather(py, [s16]) + sy[pl.ds(off, 16)]
        az = plsc.load_gather(pz, [d16]) - plsc.load_gather(pz, [s16]) + sz[pl.ds(off, 16)]
        zg = plsc.load_gather(zn, [s16])
        sb[pl.ds(off, 16)] = ax * ax + ay * ay + az * az
        db[pl.ds(off, 16)] = ax * w1 + ay * w2 + az * w3
        zb[pl.ds(off, 16)] = zg

    outs = [
        pltpu.async_copy(sb, s_out.at[pl.ds(base, EPT)], sem),
        pltpu.async_copy(db, d_out.at[pl.ds(base, EPT)], sem),
        pltpu.async_copy(zb, zs_out.at[pl.ds(base, EPT)], sem),
    ]
    for de in outs:
        de.wait()


def _silu(x):
    return 0.5 * x + (0.5 * x) * jnp.tanh(0.5 * x)


def _edge_body(s_ref, d_ref, zs_ref, W3T_ref, embT_ref, b3c_ref, W1cat_ref,
               b1_ref, cen_ref, w0_ref, W5T_ref, out_ref, nfs_ref):
    i = pl.program_id(0)

    @pl.when(i == 0)
    def _():
        x = jnp.dot(W3T_ref[...], embT_ref[...],
                    preferred_element_type=jnp.float32) + b3c_ref[...]
        nfT = _silu(x)
        hi = nfT.astype(jnp.bfloat16)
        lo = (nfT - hi.astype(jnp.float32)).astype(jnp.bfloat16)
        nfs_ref[0:D, :] = hi
        nfs_ref[D:2 * D, :] = lo

    s = s_ref[0, :]
    dd = d_ref[0, :]
    zs = zs_ref[0, :]
    se = s + 1e-12
    inv = jax.lax.rsqrt(se)
    ln = se * inv
    shw = w0_ref[0, 0] + dd * inv
    diff = cen_ref[...] - ln[None, :]
    rbf = jnp.exp(-2.0 * diff * diff)
    rh = rbf.astype(jnp.bfloat16)
    rl = (rbf - rh.astype(jnp.float32)).astype(jnp.bfloat16)
    rcat = jnp.concatenate([rh, rl], axis=0)
    pre = jnp.dot(W1cat_ref[...], rcat, preferred_element_type=jnp.float32)
    pre = pre + b1_ref[...]
    ef = _silu(pre)
    ohT = (jax.lax.broadcasted_iota(jnp.int32, (D, BE), 0) == zs[None, :])
    ohb = ohT.astype(jnp.bfloat16)
    sel = jnp.dot(nfs_ref[...], ohb, preferred_element_type=jnp.float32)
    nfT = sel[0:D, :] + sel[D:2 * D, :]
    t = nfT * ef * shw[None, :]
    out_ref[...] = jnp.dot(W5T_ref[...], t, preferred_element_type=jnp.float32)


_edge = pl.pallas_call(
    _edge_body,
    grid=(NB,),
    in_specs=[
        pl.BlockSpec((1, BE), lambda i: (0, i)),
        pl.BlockSpec((1, BE), lambda i: (0, i)),
        pl.BlockSpec((1, BE), lambda i: (0, i)),
        pl.BlockSpec((D, D), lambda i: (0, 0)),
        pl.BlockSpec((D, D), lambda i: (0, 0)),
        pl.BlockSpec((D, 1), lambda i: (0, 0)),
        pl.BlockSpec((D, 2 * NRBF), lambda i: (0, 0)),
        pl.BlockSpec((D, 1), lambda i: (0, 0)),
        pl.BlockSpec((NRBF, 1), lambda i: (0, 0)),
        pl.BlockSpec((1, 1), lambda i: (0, 0)),
        pl.BlockSpec((3, D), lambda i: (0, 0)),
    ],
    out_specs=pl.BlockSpec((3, BE), lambda i: (0, i)),
    out_shape=jax.ShapeDtypeStruct((3, E), jnp.float32),
    scratch_shapes=[pltpu.VMEM((2 * D, D), jnp.bfloat16)],
)


@functools.partial(
    pl.kernel,
    mesh=_mesh,
    compiler_params=_sc_params,
    out_type=jax.ShapeDtypeStruct((NC * ACC,), jnp.float32),
    scratch_types=[
        pltpu.VMEM((EPT,), jnp.int32),
        pltpu.VMEM((ROWS * 128,), jnp.float32),
        pltpu.VMEM((ROWS * 128,), jnp.float32),
        pltpu.VMEM((ROWS * 128,), jnp.float32),
        pltpu.VMEM((ROWS, 128), jnp.int32),
        pltpu.VMEM((ROWS, 128), jnp.int32),
        pltpu.VMEM((ROWS, 128), jnp.int32),
        pltpu.VMEM((ACC,), jnp.float32),
        pltpu.VMEM_SHARED((ACC,), jnp.float32),
        pltpu.SemaphoreType.DMA,
    ],
)
def _scat(eif, oxh, oyh, ozh, zeros_hbm, out_hbm,
          dv, ux, uy, uz, ix, iy, iz, bb, acc, sem):
    cid = lax.axis_index("c")
    sid = lax.axis_index("s")
    wid = sid * NC + cid
    base = wid * EPT
    pltpu.sync_copy(eif.at[pl.ds(E + base, EPT)], dv)
    ins = [
        pltpu.async_copy(oxh.at[pl.ds(base, EPT)], ux.at[pl.ds(0, EPT)], sem),
        pltpu.async_copy(oyh.at[pl.ds(base, EPT)], uy.at[pl.ds(0, EPT)], sem),
        pltpu.async_copy(ozh.at[pl.ds(base, EPT)], uz.at[pl.ds(0, EPT)], sem),
    ]

    @pl.when(sid == 0)
    def _():
        pltpu.sync_copy(zeros_hbm, bb)
        pltpu.sync_copy(bb, acc)

    @pl.loop(0, ROWS - 2, unroll=4)
    def ibody(r):
        for k in range(8):
            d16 = dv[pl.ds(r * 128 + k * 16, 16)]
            f = d16 * 3
            ix[r, pl.ds(k * 16, 16)] = f
            iy[r, pl.ds(k * 16, 16)] = f + 1
            iz[r, pl.ds(k * 16, 16)] = f + 2

    d16 = dv[pl.ds((ROWS - 2) * 128, 16)]
    f = d16 * 3
    ix[ROWS - 2, pl.ds(0, 16)] = f
    iy[ROWS - 2, pl.ds(0, 16)] = f + 1
    iz[ROWS - 2, pl.ds(0, 16)] = f + 2
    ar16 = jnp.arange(16, dtype=jnp.int32)
    for k in range(1, 8):
        pad = 3 * N + (k - 1) * 16 + ar16
        ix[ROWS - 2, pl.ds(k * 16, 16)] = pad
        iy[ROWS - 2, pl.ds(k * 16, 16)] = pad
        iz[ROWS - 2, pl.ds(k * 16, 16)] = pad
    for k in range(8):
        pad = 3 * N + 112 + k * 16 + ar16
        ix[ROWS - 1, pl.ds(k * 16, 16)] = pad
        iy[ROWS - 1, pl.ds(k * 16, 16)] = pad
        iz[ROWS - 1, pl.ds(k * 16, 16)] = pad

    for de in ins:
        de.wait()
    zero16 = jnp.zeros((16,), jnp.float32)
    for k in range((ROWS * 128 - EPT) // 16):
        off = EPT + k * 16
        ux[pl.ds(off, 16)] = zero16
        uy[pl.ds(off, 16)] = zero16
        uz[pl.ds(off, 16)] = zero16

    plsc.subcore_barrier()

    @pl.loop(0, ROWS // 8)
    def sbody(g):
        descs = []
        for rr in range(8):
            r = g * 8 + rr
            for u, ii in ((ux, ix), (uy, iy), (uz, iz)):
                descs.append(pltpu.async_copy(
                    u.at[pl.ds(r * 128, 128)], acc.at[ii.at[r]], sem,
                    add=True))
        for de in descs:
            de.wait()

    plsc.subcore_barrier()

    @pl.when(sid == 0)
    def _():
        pltpu.sync_copy(acc, bb)
        pltpu.sync_copy(bb, out_hbm.at[pl.ds(cid * ACC, ACC)])


def _fin_body(p_ref, z_ref, emb_ref, W4_ref, W5_ref, b5_ref, o_ref, w45_ref):
    i = pl.program_id(0)

    @pl.when(i == 0)
    def _():
        w45 = jnp.dot(W4_ref[...], W5_ref[...],
                      preferred_element_type=jnp.float32)
        w45_ref[...] = jnp.dot(emb_ref[...], w45,
                               preferred_element_type=jnp.float32)

    zb = z_ref[0, 0, :]
    oh = (jax.lax.broadcasted_iota(jnp.int32, (BN, D), 1) == zb[:, None])
    bse = jnp.dot(oh.astype(jnp.float32), w45_ref[...],
                  preferred_element_type=jnp.float32)
    o_ref[...] = p_ref[0] + p_ref[1] + bse + b5_ref[...][None, :]


_fin = pl.pallas_call(
    _fin_body,
    grid=(NBN,),
    in_specs=[
        pl.BlockSpec((2, BN, 3), lambda i: (0, i, 0)),
        pl.BlockSpec((1, 1, BN), lambda i: (i, 0, 0)),
        pl.BlockSpec((D, D), lambda i: (0, 0)),
        pl.BlockSpec((D, D), lambda i: (0, 0)),
        pl.BlockSpec((D, 3), lambda i: (0, 0)),
        pl.BlockSpec((3,), lambda i: (0,)),
    ],
    out_specs=pl.BlockSpec((BN, 3), lambda i: (i, 0)),
    out_shape=jax.ShapeDtypeStruct((N, 3), jnp.float32),
    scratch_shapes=[pltpu.VMEM((D, 3), jnp.float32)],
)


def kernel(z, pos, edge_index, shift_vector, emb, rbf_centers, W1, b1,
           W3, b3, w_sh, W4, W5, b5):
    f32 = jnp.float32
    bf16 = jnp.bfloat16
    w16 = jnp.pad(w_sh, (0, 12))
    emb128 = jnp.pad(emb, ((0, D - NT), (0, 0)))
    W1T = W1.T
    W1h = W1T.astype(bf16)
    W1l = (W1T - W1h.astype(f32)).astype(bf16)
    W1cat = jnp.concatenate([W1h, W1l], axis=1)
    eif = edge_index.reshape(2 * E)
    s_e, d_e, zs = _geom(pos[:, 0], pos[:, 1], pos[:, 2], z, eif,
                         shift_vector[:, 0], shift_vector[:, 1],
                         shift_vector[:, 2], w16)
    out3 = _edge(s_e.reshape(1, E), d_e.reshape(1, E), zs.reshape(1, E),
                 W3.T, emb128.T, b3.reshape(D, 1), W1cat,
                 b1.reshape(D, 1), rbf_centers.reshape(NRBF, 1),
                 w_sh[0].reshape(1, 1), W5.T)
    zeros = jnp.zeros((ACC,), f32)
    parts = _scat(eif, out3[0], out3[1], out3[2], zeros)
    p3 = parts.reshape(NC, ACC)[:, :3 * N].reshape(NC, N, 3)
    return _fin(p3, z.reshape(NBN, 1, BN), emb128, W4, W5, b5)
